# Initial kernel scaffold; baseline (speedup 1.0000x reference)
import jax
import jax.numpy as jnp
from jax.experimental import pallas as pl


def kernel(rel_logits, rel_pair_idxs):
    raise NotImplementedError("write your pallas kernel here")


# speedup vs baseline: 1.0065x; 1.0065x over previous
import functools

import jax
import jax.numpy as jnp
from jax.experimental import pallas as pl
from jax.experimental.pallas import tpu as pltpu

NUM_REL = 20000
NUM_CLASSES = 51
ROW_BLOCK = 2000


def _row_sum_xla_order(e):
    x = jnp.pad(e, ((0, 0), (0, (-e.shape[1]) % 8)))
    acc = x[:, 0:8]
    for k in range(1, x.shape[1] // 8):
        acc = acc + x[:, k * 8:(k + 1) * 8]
    g = 4
    while g >= 1:
        acc = acc[:, :g] + acc[:, g:2 * g]
        g //= 2
    return acc


def _softmax_body(logits_ref, prob_ref, score_ref, cls_ref):
    x = logits_ref[...]
    m = jnp.max(x, axis=1, keepdims=True)
    e = jnp.exp(x - m)
    s = _row_sum_xla_order(e)
    p = e / s
    prob_ref[...] = p
    fg = p[:, 1:]
    score_ref[0, 0, :] = jnp.max(fg, axis=1)
    cls_ref[0, 0, :] = jnp.argmax(fg, axis=1).astype(jnp.int32) + 1


def _softmax_stage(rel_logits):
    n_blocks = NUM_REL // ROW_BLOCK
    prob, score, cls = pl.pallas_call(
        _softmax_body,
        grid=(n_blocks,),
        in_specs=[pl.BlockSpec((ROW_BLOCK, NUM_CLASSES), lambda i: (i, 0))],
        out_specs=[
            pl.BlockSpec((ROW_BLOCK, NUM_CLASSES), lambda i: (i, 0)),
            pl.BlockSpec((1, 1, ROW_BLOCK), lambda i: (i, 0, 0)),
            pl.BlockSpec((1, 1, ROW_BLOCK), lambda i: (i, 0, 0)),
        ],
        out_shape=[
            jax.ShapeDtypeStruct((NUM_REL, NUM_CLASSES), jnp.float32),
            jax.ShapeDtypeStruct((n_blocks, 1, ROW_BLOCK), jnp.float32),
            jax.ShapeDtypeStruct((n_blocks, 1, ROW_BLOCK), jnp.int32),
        ],
    )(rel_logits)
    return prob, score.reshape(-1), cls.reshape(-1)


def kernel(rel_logits, rel_pair_idxs):
    prob, score, cls = _softmax_stage(rel_logits)
    idx = jnp.argsort(-score)
    return rel_pair_idxs[idx], prob[idx], cls[idx]
